# baseline (speedup 1.0000x reference)
import functools

import jax
import jax.numpy as jnp
from jax import lax
from jax.experimental import pallas as pl
from jax.experimental.pallas import tpu as pltpu

I_SIZE = 3
H = 64
OUTPAD = 128


def _gate_cols(w, layer):
    i, f, g, o = jnp.split(w, 4, axis=1)
    z = jnp.zeros_like(i)
    pairs = ((i, z), (f, z), (o, z), (g, z)) if layer == 0 else \
            ((z, i), (z, f), (z, o), (z, g))
    return jnp.concatenate([blk for pair in pairs for blk in pair], axis=1)


def _lstm_kernel(x_ref, wx_ref, wh_ref, bb_ref, wfc_ref, bfc_ref, out_ref,
                 *, T, TB):
    wx = wx_ref[...]
    wh = wh_ref[...]
    wfc = wfc_ref[...]
    bb = bb_ref[...]
    bfc = bfc_ref[...]
    lane = lax.broadcasted_iota(jnp.int32, (TB, 2 * H), 1)
    l0_mask = lane < H

    def _advance(gsum, c):
        ifo = jax.nn.sigmoid(gsum[:, :6 * H])
        gg = jnp.tanh(gsum[:, 6 * H:])
        c = ifo[:, 2 * H:4 * H] * c + ifo[:, :2 * H] * gg
        h = ifo[:, 4 * H:6 * H] * jnp.tanh(c)
        return h, c

    def _emit(h, t):
        y = jnp.dot(h, wfc, preferred_element_type=jnp.float32) + bfc
        out_ref[:, t, :] = y[:, :I_SIZE].astype(out_ref.dtype)

    g0 = jnp.dot(x_ref[:, 0, :], wx, preferred_element_type=jnp.float32) + bb
    h, c = _advance(g0, jnp.zeros((TB, 2 * H), jnp.float32))
    h = jnp.where(l0_mask, h, 0.0)
    c = jnp.where(l0_mask, c, 0.0)

    def _step(s, carry):
        h, c = carry
        gsum = (jnp.dot(h, wh, preferred_element_type=jnp.float32)
                + jnp.dot(x_ref[:, s, :], wx,
                          preferred_element_type=jnp.float32) + bb)
        h, c = _advance(gsum, c)
        _emit(h, s - 1)
        return h, c

    h, c = lax.fori_loop(1, T, _step, (h, c), unroll=2)

    gsum = jnp.dot(h, wh, preferred_element_type=jnp.float32) + bb
    h, _ = _advance(gsum, c)
    _emit(h, T - 1)


@jax.jit
def _forward(x, wih0, whh0, b0, wih1, whh1, b1, wfc, bfc):
    B, T, I = x.shape
    TB = 256
    if B < TB:
        TB = max(8, -(-B // 8) * 8)
    Bpad = -(-B // TB) * TB
    nb = Bpad // TB
    if Bpad != B:
        x = jnp.pad(x, ((0, Bpad - B), (0, 0), (0, 0)))

    bb = _gate_cols(b0, 0) + _gate_cols(b1, 1)
    wx = _gate_cols(wih0, 0)
    wh = jnp.concatenate(
        [_gate_cols(whh0, 0) + _gate_cols(wih1, 1),
         _gate_cols(whh1, 1)], axis=0)
    wfcp = jnp.zeros((2 * H, OUTPAD), jnp.float32).at[H:, :I].set(wfc)
    bfcp = jnp.zeros((1, OUTPAD), jnp.float32).at[:, :I].set(bfc)

    out = pl.pallas_call(
        functools.partial(_lstm_kernel, T=T, TB=TB),
        out_shape=jax.ShapeDtypeStruct((Bpad, T, I), x.dtype),
        grid=(nb,),
        in_specs=[
            pl.BlockSpec((TB, T, I), lambda i: (i, 0, 0)),
            pl.BlockSpec((I, 8 * H), lambda i: (0, 0)),
            pl.BlockSpec((2 * H, 8 * H), lambda i: (0, 0)),
            pl.BlockSpec((1, 8 * H), lambda i: (0, 0)),
            pl.BlockSpec((2 * H, OUTPAD), lambda i: (0, 0)),
            pl.BlockSpec((1, OUTPAD), lambda i: (0, 0)),
        ],
        out_specs=pl.BlockSpec((TB, T, I), lambda i: (i, 0, 0)),
        compiler_params=pltpu.CompilerParams(
            dimension_semantics=("parallel",),
            vmem_limit_bytes=60 * 1024 * 1024,
        ),
    )(x, wx, wh, bb, wfcp, bfcp)

    return out[:B]


def kernel(x, wih0, whh0, b0, wih1, whh1, b1, wfc, bfc):
    return _forward(x, wih0, whh0, b0, wih1, whh1, b1, wfc, bfc)


# speedup vs baseline: 1.0581x; 1.0581x over previous
import functools

import jax
import jax.numpy as jnp
from jax import lax
from jax.experimental import pallas as pl
from jax.experimental.pallas import tpu as pltpu

I_SIZE = 3
H = 64
OUTPAD = 128


def _gate_cols(w, layer):
    i, f, g, o = jnp.split(w, 4, axis=1)
    z = jnp.zeros_like(i)
    pairs = ((i, z), (f, z), (o, z), (g, z)) if layer == 0 else \
            ((z, i), (z, f), (z, o), (z, g))
    return jnp.concatenate([blk for pair in pairs for blk in pair], axis=1)


def _lstm_kernel(x_ref, wx_ref, wh_ref, bb_ref, wfc_ref, bfc_ref, out_ref,
                 pre, hseq, *, T, TB):
    wx = wx_ref[...]
    wh = wh_ref[...]
    bb = bb_ref[...]
    lane = lax.broadcasted_iota(jnp.int32, (TB, 2 * H), 1)
    l0_mask = lane < H

    for t in range(T):
        pre[pl.ds(t * TB, TB), :] = (
            jnp.dot(x_ref[:, t, :], wx, preferred_element_type=jnp.float32)
            + bb)

    def _advance(gsum, c):
        ifo = jax.nn.sigmoid(gsum[:, :6 * H])
        gg = jnp.tanh(gsum[:, 6 * H:])
        c = ifo[:, 2 * H:4 * H] * c + ifo[:, :2 * H] * gg
        h = ifo[:, 4 * H:6 * H] * jnp.tanh(c)
        return h, c

    h, c = _advance(pre[pl.ds(0, TB), :], jnp.zeros((TB, 2 * H), jnp.float32))
    h = jnp.where(l0_mask, h, 0.0)
    c = jnp.where(l0_mask, c, 0.0)

    def _step(s, carry):
        h, c = carry
        gsum = jnp.dot(h, wh, preferred_element_type=jnp.float32) \
            + pre[pl.ds(s * TB, TB), :]
        h, c = _advance(gsum, c)
        hseq[pl.ds((s - 1) * TB, TB), :] = h
        return h, c

    h, c = lax.fori_loop(1, T, _step, (h, c), unroll=2)

    gsum = jnp.dot(h, wh, preferred_element_type=jnp.float32) + bb
    h, _ = _advance(gsum, c)
    hseq[pl.ds((T - 1) * TB, TB), :] = h

    y = jnp.dot(hseq[...], wfc_ref[...],
                preferred_element_type=jnp.float32) + bfc_ref[...]
    for t in range(T):
        out_ref[:, t, :] = y[t * TB:(t + 1) * TB, :I_SIZE].astype(
            out_ref.dtype)


@jax.jit
def _forward(x, wih0, whh0, b0, wih1, whh1, b1, wfc, bfc):
    B, T, I = x.shape
    TB = 256
    if B < TB:
        TB = max(8, -(-B // 8) * 8)
    Bpad = -(-B // TB) * TB
    nb = Bpad // TB
    if Bpad != B:
        x = jnp.pad(x, ((0, Bpad - B), (0, 0), (0, 0)))

    bb = _gate_cols(b0, 0) + _gate_cols(b1, 1)
    wx = _gate_cols(wih0, 0)
    wh = jnp.concatenate(
        [_gate_cols(whh0, 0) + _gate_cols(wih1, 1),
         _gate_cols(whh1, 1)], axis=0)
    wfcp = jnp.zeros((2 * H, OUTPAD), jnp.float32).at[H:, :I].set(wfc)
    bfcp = jnp.zeros((1, OUTPAD), jnp.float32).at[:, :I].set(bfc)

    out = pl.pallas_call(
        functools.partial(_lstm_kernel, T=T, TB=TB),
        out_shape=jax.ShapeDtypeStruct((Bpad, T, I), x.dtype),
        grid=(nb,),
        in_specs=[
            pl.BlockSpec((TB, T, I), lambda i: (i, 0, 0)),
            pl.BlockSpec((I, 8 * H), lambda i: (0, 0)),
            pl.BlockSpec((2 * H, 8 * H), lambda i: (0, 0)),
            pl.BlockSpec((1, 8 * H), lambda i: (0, 0)),
            pl.BlockSpec((2 * H, OUTPAD), lambda i: (0, 0)),
            pl.BlockSpec((1, OUTPAD), lambda i: (0, 0)),
        ],
        out_specs=pl.BlockSpec((TB, T, I), lambda i: (i, 0, 0)),
        scratch_shapes=[
            pltpu.VMEM((T * TB, 8 * H), jnp.float32),
            pltpu.VMEM((T * TB, 2 * H), jnp.float32),
        ],
        compiler_params=pltpu.CompilerParams(
            dimension_semantics=("parallel",),
            vmem_limit_bytes=60 * 1024 * 1024,
        ),
    )(x, wx, wh, bb, wfcp, bfcp)

    return out[:B]


def kernel(x, wih0, whh0, b0, wih1, whh1, b1, wfc, bfc):
    return _forward(x, wih0, whh0, b0, wih1, whh1, b1, wfc, bfc)


# speedup vs baseline: 1.1269x; 1.0650x over previous
import functools

import jax
import jax.numpy as jnp
from jax import lax
from jax.experimental import pallas as pl
from jax.experimental.pallas import tpu as pltpu

I_SIZE = 3
H = 64
OUTPAD = 128


def _gate_cols(w, layer):
    i, f, g, o = jnp.split(w, 4, axis=1)
    z = jnp.zeros_like(i)
    pairs = ((i, z), (f, z), (o, z), (g, z)) if layer == 0 else \
            ((z, i), (z, f), (z, o), (z, g))
    return jnp.concatenate([blk for pair in pairs for blk in pair], axis=1)


def _lstm_kernel(x_ref, wx_ref, wh_ref, bb_ref, wfc_ref, bfc_ref, out_ref,
                 pre, hseq, *, T, TB):
    wx = wx_ref[...]
    wh = wh_ref[...]
    bb = bb_ref[...]
    lane = lax.broadcasted_iota(jnp.int32, (TB, 2 * H), 1)
    l0_mask = lane < H

    xb = x_ref[...].astype(jnp.bfloat16)
    for t in range(T):
        pre[pl.ds(t * TB, TB), :] = (
            jnp.dot(xb[:, t, :], wx, preferred_element_type=jnp.float32)
            + bb).astype(jnp.bfloat16)

    def _advance(gsum, c):
        ifo = jax.nn.sigmoid(gsum[:, :6 * H])
        gg = jnp.tanh(gsum[:, 6 * H:])
        c = ifo[:, 2 * H:4 * H] * c + ifo[:, :2 * H] * gg
        h = ifo[:, 4 * H:6 * H] * jnp.tanh(c)
        return h, c

    h, c = _advance(pre[pl.ds(0, TB), :].astype(jnp.float32),
                    jnp.zeros((TB, 2 * H), jnp.float32))
    h = jnp.where(l0_mask, h, 0.0)
    c = jnp.where(l0_mask, c, 0.0)

    def _step(s, carry):
        h, c = carry
        gsum = jnp.dot(h.astype(jnp.bfloat16), wh,
                       preferred_element_type=jnp.float32) \
            + pre[pl.ds(s * TB, TB), :].astype(jnp.float32)
        h, c = _advance(gsum, c)
        hseq[pl.ds((s - 1) * TB, TB), :] = h.astype(jnp.bfloat16)
        return h, c

    h, c = lax.fori_loop(1, T, _step, (h, c), unroll=2)

    gsum = jnp.dot(h.astype(jnp.bfloat16), wh,
                   preferred_element_type=jnp.float32) + bb
    h, _ = _advance(gsum, c)
    hseq[pl.ds((T - 1) * TB, TB), :] = h.astype(jnp.bfloat16)

    y = jnp.dot(hseq[...], wfc_ref[...],
                preferred_element_type=jnp.float32) + bfc_ref[...]
    for t in range(T):
        out_ref[:, t, :] = y[t * TB:(t + 1) * TB, :I_SIZE].astype(
            out_ref.dtype)


@jax.jit
def _forward(x, wih0, whh0, b0, wih1, whh1, b1, wfc, bfc):
    B, T, I = x.shape
    TB = 256
    if B < TB:
        TB = max(8, -(-B // 8) * 8)
    Bpad = -(-B // TB) * TB
    nb = Bpad // TB
    if Bpad != B:
        x = jnp.pad(x, ((0, Bpad - B), (0, 0), (0, 0)))

    bb = _gate_cols(b0, 0) + _gate_cols(b1, 1)
    wx = _gate_cols(wih0, 0).astype(jnp.bfloat16)
    wh = jnp.concatenate(
        [_gate_cols(whh0, 0) + _gate_cols(wih1, 1),
         _gate_cols(whh1, 1)], axis=0).astype(jnp.bfloat16)
    wfcp = jnp.zeros((2 * H, OUTPAD), jnp.float32).at[H:, :I].set(wfc)
    wfcp = wfcp.astype(jnp.bfloat16)
    bfcp = jnp.zeros((1, OUTPAD), jnp.float32).at[:, :I].set(bfc)

    out = pl.pallas_call(
        functools.partial(_lstm_kernel, T=T, TB=TB),
        out_shape=jax.ShapeDtypeStruct((Bpad, T, I), x.dtype),
        grid=(nb,),
        in_specs=[
            pl.BlockSpec((TB, T, I), lambda i: (i, 0, 0)),
            pl.BlockSpec((I, 8 * H), lambda i: (0, 0)),
            pl.BlockSpec((2 * H, 8 * H), lambda i: (0, 0)),
            pl.BlockSpec((1, 8 * H), lambda i: (0, 0)),
            pl.BlockSpec((2 * H, OUTPAD), lambda i: (0, 0)),
            pl.BlockSpec((1, OUTPAD), lambda i: (0, 0)),
        ],
        out_specs=pl.BlockSpec((TB, T, I), lambda i: (i, 0, 0)),
        scratch_shapes=[
            pltpu.VMEM((T * TB, 8 * H), jnp.bfloat16),
            pltpu.VMEM((T * TB, 2 * H), jnp.bfloat16),
        ],
        compiler_params=pltpu.CompilerParams(
            dimension_semantics=("parallel",),
            vmem_limit_bytes=60 * 1024 * 1024,
        ),
    )(x, wx, wh, bb, wfcp, bfcp)

    return out[:B]


def kernel(x, wih0, whh0, b0, wih1, whh1, b1, wfc, bfc):
    return _forward(x, wih0, whh0, b0, wih1, whh1, b1, wfc, bfc)


# speedup vs baseline: 1.2762x; 1.1325x over previous
import functools

import jax
import jax.numpy as jnp
from jax import lax
from jax.experimental import pallas as pl
from jax.experimental.pallas import tpu as pltpu

I_SIZE = 3
H = 64
OUTPAD = 128


def _gate_cols(w, layer):
    i, f, g, o = jnp.split(w, 4, axis=1)
    z = jnp.zeros_like(i)
    pairs = ((i, z), (f, z), (o, z), (g, z)) if layer == 0 else \
            ((z, i), (z, f), (z, o), (z, g))
    return jnp.concatenate([blk for pair in pairs for blk in pair], axis=1)


def _lstm_kernel(x_ref, wx_ref, wh_ref, bb_ref, wfc_ref, bfc_ref, out_ref,
                 pre, hseq, *, T, TB):
    wx = wx_ref[...]
    wh = wh_ref[...]
    bb = bb_ref[...]
    lane = lax.broadcasted_iota(jnp.int32, (TB, 2 * H), 1)
    l0_mask = lane < H

    xb = x_ref[...]
    for t in range(T):
        pre[pl.ds(t * TB, TB), :] = (
            jnp.dot(xb[:, t, :], wx, preferred_element_type=jnp.float32)
            + bb).astype(jnp.bfloat16)

    def _advance(gsum, c):
        ifo = jax.nn.sigmoid(gsum[:, :6 * H])
        gg = jnp.tanh(gsum[:, 6 * H:])
        c = ifo[:, 2 * H:4 * H] * c + ifo[:, :2 * H] * gg
        h = ifo[:, 4 * H:6 * H] * jnp.tanh(c)
        return h, c

    h, c = _advance(pre[pl.ds(0, TB), :].astype(jnp.float32),
                    jnp.zeros((TB, 2 * H), jnp.float32))
    h = jnp.where(l0_mask, h, 0.0)
    c = jnp.where(l0_mask, c, 0.0)

    def _step(s, carry):
        h, c = carry
        gsum = jnp.dot(h.astype(jnp.bfloat16), wh,
                       preferred_element_type=jnp.float32) \
            + pre[pl.ds(s * TB, TB), :].astype(jnp.float32)
        h, c = _advance(gsum, c)
        hseq[pl.ds((s - 1) * TB, TB), :] = h.astype(jnp.bfloat16)
        return h, c

    h, c = lax.fori_loop(1, T, _step, (h, c), unroll=2)

    gsum = jnp.dot(h.astype(jnp.bfloat16), wh,
                   preferred_element_type=jnp.float32) + bb
    h, _ = _advance(gsum, c)
    hseq[pl.ds((T - 1) * TB, TB), :] = h.astype(jnp.bfloat16)

    y = jnp.dot(hseq[...], wfc_ref[...],
                preferred_element_type=jnp.float32) + bfc_ref[...]
    for t in range(T):
        out_ref[:, t, :] = y[t * TB:(t + 1) * TB, :I_SIZE].astype(
            out_ref.dtype)


@jax.jit
def _forward(x, wih0, whh0, b0, wih1, whh1, b1, wfc, bfc):
    B, T, I = x.shape
    TB = 512
    if B < TB:
        TB = max(8, -(-B // 8) * 8)
    Bpad = -(-B // TB) * TB
    nb = Bpad // TB
    xb = x.astype(jnp.bfloat16)
    if Bpad != B:
        xb = jnp.pad(xb, ((0, Bpad - B), (0, 0), (0, 0)))

    bb = _gate_cols(b0, 0) + _gate_cols(b1, 1)
    wx = _gate_cols(wih0, 0).astype(jnp.bfloat16)
    wh = jnp.concatenate(
        [_gate_cols(whh0, 0) + _gate_cols(wih1, 1),
         _gate_cols(whh1, 1)], axis=0).astype(jnp.bfloat16)
    wfcp = jnp.zeros((2 * H, OUTPAD), jnp.float32).at[H:, :I].set(wfc)
    wfcp = wfcp.astype(jnp.bfloat16)
    bfcp = jnp.zeros((1, OUTPAD), jnp.float32).at[:, :I].set(bfc)

    out = pl.pallas_call(
        functools.partial(_lstm_kernel, T=T, TB=TB),
        out_shape=jax.ShapeDtypeStruct((Bpad, T, I), jnp.float32),
        grid=(nb,),
        in_specs=[
            pl.BlockSpec((TB, T, I), lambda i: (i, 0, 0)),
            pl.BlockSpec((I, 8 * H), lambda i: (0, 0)),
            pl.BlockSpec((2 * H, 8 * H), lambda i: (0, 0)),
            pl.BlockSpec((1, 8 * H), lambda i: (0, 0)),
            pl.BlockSpec((2 * H, OUTPAD), lambda i: (0, 0)),
            pl.BlockSpec((1, OUTPAD), lambda i: (0, 0)),
        ],
        out_specs=pl.BlockSpec((TB, T, I), lambda i: (i, 0, 0)),
        scratch_shapes=[
            pltpu.VMEM((T * TB, 8 * H), jnp.bfloat16),
            pltpu.VMEM((T * TB, 2 * H), jnp.bfloat16),
        ],
        compiler_params=pltpu.CompilerParams(
            dimension_semantics=("parallel",),
            vmem_limit_bytes=60 * 1024 * 1024,
        ),
    )(xb, wx, wh, bb, wfcp, bfcp)

    return out[:B]


def kernel(x, wih0, whh0, b0, wih1, whh1, b1, wfc, bfc):
    return _forward(x, wih0, whh0, b0, wih1, whh1, b1, wfc, bfc)


# speedup vs baseline: 2.2979x; 1.8006x over previous
import functools

import jax
import jax.numpy as jnp
from jax import lax
from jax.experimental import pallas as pl
from jax.experimental.pallas import tpu as pltpu

I_SIZE = 3
H = 64
G = 8 * H
OUTPAD = 128


def _gate_cols(w, layer):
    i, f, g, o = jnp.split(w, 4, axis=1)
    z = jnp.zeros_like(i)
    pairs = ((i, z), (f, z), (o, z), (g, z)) if layer == 0 else \
            ((z, i), (z, f), (z, o), (z, g))
    return jnp.concatenate([blk for pair in pairs for blk in pair], axis=1)


def _lstm_kernel(xr_ref, wxb_ref, wh_ref, bbt_ref, bb_ref, wfcb_ref, bfct_ref,
                 out_ref, pre, hseq, *, T, TB):
    pre[...] = (jnp.dot(xr_ref[...], wxb_ref[...],
                        preferred_element_type=jnp.float32)
                + bbt_ref[...]).astype(jnp.bfloat16)

    wh = wh_ref[...]
    lane = lax.broadcasted_iota(jnp.int32, (TB, 2 * H), 1)
    l0_mask = lane < H

    def _advance(gsum, c):
        ifo = jax.nn.sigmoid(gsum[:, :6 * H])
        gg = jnp.tanh(gsum[:, 6 * H:])
        c = ifo[:, 2 * H:4 * H] * c + ifo[:, :2 * H] * gg
        h = ifo[:, 4 * H:6 * H] * jnp.tanh(c)
        return h, c

    h, c = _advance(pre[:, :G].astype(jnp.float32),
                    jnp.zeros((TB, 2 * H), jnp.float32))
    h = jnp.where(l0_mask, h, 0.0).astype(jnp.bfloat16)
    c = jnp.where(l0_mask, c, 0.0)

    for s in range(1, T + 1):
        gsum = jnp.dot(h, wh, preferred_element_type=jnp.float32)
        if s < T:
            gsum = gsum + pre[:, s * G:(s + 1) * G].astype(jnp.float32)
        else:
            gsum = gsum + bb_ref[...]
        h, c = _advance(gsum, c)
        h = h.astype(jnp.bfloat16)
        hseq[:, (s - 1) * 2 * H:s * 2 * H] = h

    y = jnp.dot(hseq[...], wfcb_ref[...],
                preferred_element_type=jnp.float32) + bfct_ref[...]
    out_ref[...] = y[:, :T * I_SIZE].astype(out_ref.dtype)


@jax.jit
def _forward(x, wih0, whh0, b0, wih1, whh1, b1, wfc, bfc):
    B, T, I = x.shape
    TB = 512
    if B < TB:
        TB = max(8, -(-B // 8) * 8)
    Bpad = -(-B // TB) * TB
    nb = Bpad // TB
    xr = x.astype(jnp.bfloat16).reshape(B, T * I)
    if Bpad != B:
        xr = jnp.pad(xr, ((0, Bpad - B), (0, 0)))

    bb = _gate_cols(b0, 0) + _gate_cols(b1, 1)
    wx = _gate_cols(wih0, 0)
    wh = jnp.concatenate(
        [_gate_cols(whh0, 0) + _gate_cols(wih1, 1),
         _gate_cols(whh1, 1)], axis=0).astype(jnp.bfloat16)

    wxb = jnp.zeros((T * I, T * G), jnp.float32)
    for t in range(T):
        wxb = wxb.at[t * I:(t + 1) * I, t * G:(t + 1) * G].set(wx)
    wxb = wxb.astype(jnp.bfloat16)
    bbt = jnp.tile(bb, (1, T))

    wfcb = jnp.zeros((T * 2 * H, OUTPAD), jnp.float32)
    for t in range(T):
        wfcb = wfcb.at[t * 2 * H + H:(t + 1) * 2 * H, t * I:(t + 1) * I].set(wfc)
    wfcb = wfcb.astype(jnp.bfloat16)
    bfct = jnp.zeros((1, OUTPAD), jnp.float32).at[:, :T * I].set(
        jnp.tile(bfc, (1, T)))

    out = pl.pallas_call(
        functools.partial(_lstm_kernel, T=T, TB=TB),
        out_shape=jax.ShapeDtypeStruct((Bpad, T * I), jnp.float32),
        grid=(nb,),
        in_specs=[
            pl.BlockSpec((TB, T * I), lambda i: (i, 0)),
            pl.BlockSpec((T * I, T * G), lambda i: (0, 0)),
            pl.BlockSpec((2 * H, G), lambda i: (0, 0)),
            pl.BlockSpec((1, T * G), lambda i: (0, 0)),
            pl.BlockSpec((1, G), lambda i: (0, 0)),
            pl.BlockSpec((T * 2 * H, OUTPAD), lambda i: (0, 0)),
            pl.BlockSpec((1, OUTPAD), lambda i: (0, 0)),
        ],
        out_specs=pl.BlockSpec((TB, T * I), lambda i: (i, 0)),
        scratch_shapes=[
            pltpu.VMEM((TB, T * G), jnp.bfloat16),
            pltpu.VMEM((TB, T * 2 * H), jnp.bfloat16),
        ],
        compiler_params=pltpu.CompilerParams(
            dimension_semantics=("parallel",),
            vmem_limit_bytes=60 * 1024 * 1024,
        ),
    )(xr, wxb, wh, bbt, bb, wfcb, bfct)

    return out[:B].reshape(B, T, I)


def kernel(x, wih0, whh0, b0, wih1, whh1, b1, wfc, bfc):
    return _forward(x, wih0, whh0, b0, wih1, whh1, b1, wfc, bfc)


# speedup vs baseline: 2.4632x; 1.0720x over previous
import functools

import jax
import jax.numpy as jnp
from jax import lax
from jax.experimental import pallas as pl
from jax.experimental.pallas import tpu as pltpu

I_SIZE = 3
H = 64
G = 8 * H
OUTPAD = 128


def _gate_cols(w, layer):
    i, f, g, o = jnp.split(w, 4, axis=1)
    z = jnp.zeros_like(i)
    pairs = ((i, z), (f, z), (o, z), (g, z)) if layer == 0 else \
            ((z, i), (z, f), (z, o), (z, g))
    return jnp.concatenate([blk for pair in pairs for blk in pair], axis=1)


def _lstm_kernel(xr_ref, wxb_ref, wh_ref, bbt_ref, bb_ref, wfcb_ref, bfct_ref,
                 out_ref, pre, hseq, *, T, TB):
    pre[...] = jnp.dot(xr_ref[...], wxb_ref[...],
                       preferred_element_type=jnp.float32) + bbt_ref[...]

    wh = wh_ref[...]
    lane = lax.broadcasted_iota(jnp.int32, (TB, 2 * H), 1)
    l0_mask = lane < H

    def _advance(gsum, c):
        ifo = 0.5 * jnp.tanh(0.5 * gsum[:, :6 * H]) + 0.5
        gg = jnp.tanh(gsum[:, 6 * H:])
        c = ifo[:, 2 * H:4 * H] * c + ifo[:, :2 * H] * gg
        h = ifo[:, 4 * H:6 * H] * jnp.tanh(c)
        return h, c

    h, c = _advance(pre[:, :G], jnp.zeros((TB, 2 * H), jnp.float32))
    h = jnp.where(l0_mask, h, 0.0).astype(jnp.bfloat16)
    c = jnp.where(l0_mask, c, 0.0)

    for s in range(1, T + 1):
        gsum = jnp.dot(h, wh, preferred_element_type=jnp.float32)
        if s < T:
            gsum = gsum + pre[:, s * G:(s + 1) * G]
        else:
            gsum = gsum + bb_ref[...]
        h, c = _advance(gsum, c)
        h = h.astype(jnp.bfloat16)
        hseq[:, (s - 1) * 2 * H:s * 2 * H] = h

    y = jnp.dot(hseq[...], wfcb_ref[...],
                preferred_element_type=jnp.float32) + bfct_ref[...]
    out_ref[...] = y[:, :T * I_SIZE].astype(out_ref.dtype)


@jax.jit
def _forward(x, wih0, whh0, b0, wih1, whh1, b1, wfc, bfc):
    B, T, I = x.shape
    TB = 512
    if B < TB:
        TB = max(8, -(-B // 8) * 8)
    Bpad = -(-B // TB) * TB
    nb = Bpad // TB
    xr = x.astype(jnp.bfloat16).reshape(B, T * I)
    if Bpad != B:
        xr = jnp.pad(xr, ((0, Bpad - B), (0, 0)))

    bb = _gate_cols(b0, 0) + _gate_cols(b1, 1)
    wx = _gate_cols(wih0, 0)
    wh = jnp.concatenate(
        [_gate_cols(whh0, 0) + _gate_cols(wih1, 1),
         _gate_cols(whh1, 1)], axis=0).astype(jnp.bfloat16)

    wxb = jnp.zeros((T * I, T * G), jnp.float32)
    for t in range(T):
        wxb = wxb.at[t * I:(t + 1) * I, t * G:(t + 1) * G].set(wx)
    wxb = wxb.astype(jnp.bfloat16)
    bbt = jnp.tile(bb, (1, T))

    wfcb = jnp.zeros((T * 2 * H, OUTPAD), jnp.float32)
    for t in range(T):
        wfcb = wfcb.at[t * 2 * H + H:(t + 1) * 2 * H, t * I:(t + 1) * I].set(wfc)
    wfcb = wfcb.astype(jnp.bfloat16)
    bfct = jnp.zeros((1, OUTPAD), jnp.float32).at[:, :T * I].set(
        jnp.tile(bfc, (1, T)))

    out = pl.pallas_call(
        functools.partial(_lstm_kernel, T=T, TB=TB),
        out_shape=jax.ShapeDtypeStruct((Bpad, T * I), jnp.float32),
        grid=(nb,),
        in_specs=[
            pl.BlockSpec((TB, T * I), lambda i: (i, 0)),
            pl.BlockSpec((T * I, T * G), lambda i: (0, 0)),
            pl.BlockSpec((2 * H, G), lambda i: (0, 0)),
            pl.BlockSpec((1, T * G), lambda i: (0, 0)),
            pl.BlockSpec((1, G), lambda i: (0, 0)),
            pl.BlockSpec((T * 2 * H, OUTPAD), lambda i: (0, 0)),
            pl.BlockSpec((1, OUTPAD), lambda i: (0, 0)),
        ],
        out_specs=pl.BlockSpec((TB, T * I), lambda i: (i, 0)),
        scratch_shapes=[
            pltpu.VMEM((TB, T * G), jnp.float32),
            pltpu.VMEM((TB, T * 2 * H), jnp.bfloat16),
        ],
        compiler_params=pltpu.CompilerParams(
            dimension_semantics=("parallel",),
            vmem_limit_bytes=60 * 1024 * 1024,
        ),
    )(xr, wxb, wh, bbt, bb, wfcb, bfct)

    return out[:B].reshape(B, T, I)


def kernel(x, wih0, whh0, b0, wih1, whh1, b1, wfc, bfc):
    return _forward(x, wih0, whh0, b0, wih1, whh1, b1, wfc, bfc)


# speedup vs baseline: 3.4753x; 1.4109x over previous
import functools

import jax
import jax.numpy as jnp
from jax import lax
from jax.experimental import pallas as pl
from jax.experimental.pallas import tpu as pltpu

I_SIZE = 3
H = 64
G = 8 * H
SLOT = 2 * H
OUTPAD = 128


def _gate_cols(w, layer):
    i, f, g, o = jnp.split(w, 4, axis=1)
    z = jnp.zeros_like(i)
    pairs = ((i, z), (f, z), (o, z), (g, z)) if layer == 0 else \
            ((z, i), (z, f), (z, o), (z, g))
    return jnp.concatenate([blk for pair in pairs for blk in pair], axis=1)


def _lstm_kernel(xr_ref, p_ref, ones_ref, w_ref, wfcb_ref, bfct_ref,
                 out_ref, xu, hseq, *, T, TB):
    xu[...] = (jnp.dot(xr_ref[...], p_ref[...],
                       preferred_element_type=jnp.float32)
               + ones_ref[...]).astype(jnp.bfloat16)

    w = w_ref[...]
    lane = lax.broadcasted_iota(jnp.int32, (TB, 2 * H), 1)
    l0_mask = lane < H

    h = jnp.zeros((TB, 2 * H), jnp.bfloat16)
    c = jnp.zeros((TB, 2 * H), jnp.float32)

    for s in range(T + 1):
        hx = jnp.concatenate([h, xu[:, s * SLOT:(s + 1) * SLOT]], axis=1)
        z = jnp.dot(hx, w, preferred_element_type=jnp.float32)
        t = jnp.tanh(z)
        ti = t[:, :2 * H]
        tf = t[:, 2 * H:4 * H]
        to = t[:, 4 * H:6 * H]
        tg = t[:, 6 * H:]
        c = 0.5 * ((tf * c + c) + (ti * tg + tg))
        tc = jnp.tanh(c)
        hf = 0.5 * (to * tc + tc)
        if s == 0:
            hf = jnp.where(l0_mask, hf, 0.0)
            c = jnp.where(l0_mask, c, 0.0)
            h = hf.astype(jnp.bfloat16)
        else:
            h = hf.astype(jnp.bfloat16)
            hseq[:, (s - 1) * 2 * H:s * 2 * H] = h

    y = jnp.dot(hseq[...], wfcb_ref[...],
                preferred_element_type=jnp.float32) + bfct_ref[...]
    out_ref[...] = y[:, :T * I_SIZE].astype(out_ref.dtype)


@jax.jit
def _forward(x, wih0, whh0, b0, wih1, whh1, b1, wfc, bfc):
    B, T, I = x.shape
    TB = 512
    if B < TB:
        TB = max(8, -(-B // 8) * 8)
    Bpad = -(-B // TB) * TB
    nb = Bpad // TB
    xr = x.astype(jnp.bfloat16).reshape(B, T * I)
    if Bpad != B:
        xr = jnp.pad(xr, ((0, Bpad - B), (0, 0)))

    bb = _gate_cols(b0, 0) + _gate_cols(b1, 1)
    wx = _gate_cols(wih0, 0)
    wh = jnp.concatenate(
        [_gate_cols(whh0, 0) + _gate_cols(wih1, 1),
         _gate_cols(whh1, 1)], axis=0)

    wxe = jnp.zeros((SLOT, G), jnp.float32)
    wxe = wxe.at[:I, :].set(wx).at[I, :].set(bb[0])
    w = jnp.concatenate([wh, wxe], axis=0)
    w = w.at[:, :6 * H].multiply(0.5).astype(jnp.bfloat16)

    NS = T + 1
    p = jnp.zeros((T * I, NS * SLOT), jnp.float32)
    for t in range(T):
        p = p.at[t * I:(t + 1) * I, t * SLOT:t * SLOT + I].set(jnp.eye(I))
    p = p.astype(jnp.bfloat16)
    ones_row = jnp.zeros((1, NS * SLOT), jnp.float32)
    ones_row = ones_row.at[0, jnp.arange(NS) * SLOT + I].set(1.0)

    wfcb = jnp.zeros((T * 2 * H, OUTPAD), jnp.float32)
    for t in range(T):
        wfcb = wfcb.at[t * 2 * H + H:(t + 1) * 2 * H, t * I:(t + 1) * I].set(wfc)
    wfcb = wfcb.astype(jnp.bfloat16)
    bfct = jnp.zeros((1, OUTPAD), jnp.float32).at[:, :T * I].set(
        jnp.tile(bfc, (1, T)))

    out = pl.pallas_call(
        functools.partial(_lstm_kernel, T=T, TB=TB),
        out_shape=jax.ShapeDtypeStruct((Bpad, T * I), jnp.float32),
        grid=(nb,),
        in_specs=[
            pl.BlockSpec((TB, T * I), lambda i: (i, 0)),
            pl.BlockSpec((T * I, NS * SLOT), lambda i: (0, 0)),
            pl.BlockSpec((1, NS * SLOT), lambda i: (0, 0)),
            pl.BlockSpec((2 * H + SLOT, G), lambda i: (0, 0)),
            pl.BlockSpec((T * 2 * H, OUTPAD), lambda i: (0, 0)),
            pl.BlockSpec((1, OUTPAD), lambda i: (0, 0)),
        ],
        out_specs=pl.BlockSpec((TB, T * I), lambda i: (i, 0)),
        scratch_shapes=[
            pltpu.VMEM((TB, NS * SLOT), jnp.bfloat16),
            pltpu.VMEM((TB, T * 2 * H), jnp.bfloat16),
        ],
        compiler_params=pltpu.CompilerParams(
            dimension_semantics=("parallel",),
            vmem_limit_bytes=60 * 1024 * 1024,
        ),
    )(xr, p, ones_row, w, wfcb, bfct)

    return out[:B].reshape(B, T, I)


def kernel(x, wih0, whh0, b0, wih1, whh1, b1, wfc, bfc):
    return _forward(x, wih0, whh0, b0, wih1, whh1, b1, wfc, bfc)


# speedup vs baseline: 3.5007x; 1.0073x over previous
import functools

import jax
import jax.numpy as jnp
from jax import lax
from jax.experimental import pallas as pl
from jax.experimental.pallas import tpu as pltpu

I_SIZE = 3
H = 64
G = 8 * H
SLOT = 2 * H
OUTPAD = 128


def _gate_cols(w, layer):
    i, f, g, o = jnp.split(w, 4, axis=1)
    z = jnp.zeros_like(i)
    pairs = ((i, z), (f, z), (o, z), (g, z)) if layer == 0 else \
            ((z, i), (z, f), (z, o), (z, g))
    return jnp.concatenate([blk for pair in pairs for blk in pair], axis=1)


def _lstm_kernel(xr_ref, p_ref, ones_ref, w_ref, wfcb_ref, bfct_ref,
                 out_ref, xu, hseq, *, T, TB):
    xu[...] = (jnp.dot(xr_ref[...], p_ref[...],
                       preferred_element_type=jnp.float32)
               + ones_ref[...]).astype(jnp.bfloat16)

    w = w_ref[...]
    lane = lax.broadcasted_iota(jnp.int32, (TB, 2 * H), 1)
    l0_mask = lane < H

    h = jnp.zeros((TB, 2 * H), jnp.bfloat16)
    c = jnp.zeros((TB, 2 * H), jnp.float32)

    for s in range(T + 1):
        hx = jnp.concatenate([h, xu[:, s * SLOT:(s + 1) * SLOT]], axis=1)
        z = jnp.dot(hx, w, preferred_element_type=jnp.float32)
        t = jnp.tanh(z)
        ti = t[:, :2 * H]
        tf = t[:, 2 * H:4 * H]
        to = t[:, 4 * H:6 * H]
        tg = t[:, 6 * H:]
        c = 0.5 * ((tf * c + c) + (ti * tg + tg))
        tc = jnp.tanh(c)
        hf = 0.5 * (to * tc + tc)
        if s == 0:
            hf = jnp.where(l0_mask, hf, 0.0)
            c = jnp.where(l0_mask, c, 0.0)
            h = hf.astype(jnp.bfloat16)
        else:
            h = hf.astype(jnp.bfloat16)
            hseq[:, (s - 1) * 2 * H:s * 2 * H] = h

    y = jnp.dot(hseq[...], wfcb_ref[...],
                preferred_element_type=jnp.float32) + bfct_ref[...]
    out_ref[...] = y[:, :T * I_SIZE].astype(out_ref.dtype)


@jax.jit
def _forward(x, wih0, whh0, b0, wih1, whh1, b1, wfc, bfc):
    B, T, I = x.shape
    TB = 1024
    if B < TB:
        TB = max(8, -(-B // 8) * 8)
    Bpad = -(-B // TB) * TB
    nb = Bpad // TB
    xr = x.astype(jnp.bfloat16).reshape(B, T * I)
    if Bpad != B:
        xr = jnp.pad(xr, ((0, Bpad - B), (0, 0)))

    bb = _gate_cols(b0, 0) + _gate_cols(b1, 1)
    wx = _gate_cols(wih0, 0)
    wh = jnp.concatenate(
        [_gate_cols(whh0, 0) + _gate_cols(wih1, 1),
         _gate_cols(whh1, 1)], axis=0)

    wxe = jnp.zeros((SLOT, G), jnp.float32)
    wxe = wxe.at[:I, :].set(wx).at[I, :].set(bb[0])
    w = jnp.concatenate([wh, wxe], axis=0)
    w = w.at[:, :6 * H].multiply(0.5).astype(jnp.bfloat16)

    NS = T + 1
    p = jnp.zeros((T * I, NS * SLOT), jnp.float32)
    for t in range(T):
        p = p.at[t * I:(t + 1) * I, t * SLOT:t * SLOT + I].set(jnp.eye(I))
    p = p.astype(jnp.bfloat16)
    ones_row = jnp.zeros((1, NS * SLOT), jnp.float32)
    ones_row = ones_row.at[0, jnp.arange(NS) * SLOT + I].set(1.0)

    wfcb = jnp.zeros((T * 2 * H, OUTPAD), jnp.float32)
    for t in range(T):
        wfcb = wfcb.at[t * 2 * H + H:(t + 1) * 2 * H, t * I:(t + 1) * I].set(wfc)
    wfcb = wfcb.astype(jnp.bfloat16)
    bfct = jnp.zeros((1, OUTPAD), jnp.float32).at[:, :T * I].set(
        jnp.tile(bfc, (1, T)))

    out = pl.pallas_call(
        functools.partial(_lstm_kernel, T=T, TB=TB),
        out_shape=jax.ShapeDtypeStruct((Bpad, T * I), jnp.float32),
        grid=(nb,),
        in_specs=[
            pl.BlockSpec((TB, T * I), lambda i: (i, 0)),
            pl.BlockSpec((T * I, NS * SLOT), lambda i: (0, 0)),
            pl.BlockSpec((1, NS * SLOT), lambda i: (0, 0)),
            pl.BlockSpec((2 * H + SLOT, G), lambda i: (0, 0)),
            pl.BlockSpec((T * 2 * H, OUTPAD), lambda i: (0, 0)),
            pl.BlockSpec((1, OUTPAD), lambda i: (0, 0)),
        ],
        out_specs=pl.BlockSpec((TB, T * I), lambda i: (i, 0)),
        scratch_shapes=[
            pltpu.VMEM((TB, NS * SLOT), jnp.bfloat16),
            pltpu.VMEM((TB, T * 2 * H), jnp.bfloat16),
        ],
        compiler_params=pltpu.CompilerParams(
            dimension_semantics=("parallel",),
            vmem_limit_bytes=60 * 1024 * 1024,
        ),
    )(xr, p, ones_row, w, wfcb, bfct)

    return out[:B].reshape(B, T, I)


def kernel(x, wih0, whh0, b0, wih1, whh1, b1, wfc, bfc):
    return _forward(x, wih0, whh0, b0, wih1, whh1, b1, wfc, bfc)
